# baseline (speedup 1.0000x reference)
import functools

import jax
import jax.numpy as jnp
from jax import lax
from jax.experimental import pallas as pl
from jax.experimental.pallas import tpu as pltpu
from jax.experimental.pallas import tpu_sc as plsc

N = 10000
E = 320000
F_IN = 128
H = 128
C = 64

NC = 2
NS = 16
NW = NC * NS
EPT = E // NW
CHUNKP = 80
NCHUNK = EPT // CHUNKP
EPT2 = E // NS
NCHUNK2 = EPT2 // CHUNKP
ROW0 = 624
TAIL0 = N - NS * ROW0
ZR = 40
NBUF = 5
ROW_BLK = 2000


def _sc_mesh():
    return plsc.VectorSubcoreMesh(
        core_axis_name="c", subcore_axis_name="s",
        num_cores=NC, num_subcores=NS)


def _fill(ref, rows, width, value):
    vec = jnp.full((16,), value, jnp.float32)

    @pl.loop(0, rows)
    def _(r):
        for c in range(0, width, 16):
            ref[r, pl.ds(c, 16)] = vec


def _zero_acc(zb, acc, sid, sem):
    base = sid * ROW0
    for z in range(16):
        pltpu.async_copy(zb, acc.at[pl.ds(base + z * ZR, ZR)], sem).wait()


def _read_out(acc, out_hbm, cid, sid, sem):
    base = sid * ROW0
    pltpu.async_copy(acc.at[pl.ds(base, ROW0)],
                     out_hbm.at[cid, pl.ds(base, ROW0)], sem).wait()

    @pl.when(sid == NS - 1)
    def _():
        t0 = NS * ROW0
        pltpu.async_copy(acc.at[pl.ds(t0, TAIL0)],
                         out_hbm.at[cid, pl.ds(t0, TAIL0)], sem).wait()


def _sc_degree(dst):
    @functools.partial(
        pl.kernel,
        out_type=jax.ShapeDtypeStruct((NC, N, 16), jnp.float32),
        mesh=_sc_mesh(),
        scratch_types=[
            [pltpu.VMEM((CHUNKP,), jnp.int32)] * NBUF,
            pltpu.VMEM((CHUNKP, 16), jnp.float32),
            pltpu.VMEM((ZR, 16), jnp.float32),
            pltpu.VMEM_SHARED((N, 16), jnp.float32),
            [pltpu.SemaphoreType.DMA] * NBUF,
            pltpu.SemaphoreType.DMA,
        ])
    def deg_kernel(dst_hbm, out_hbm, didxs, ones_v, zb, acc, ssems, zsem):
        cid = lax.axis_index("c")
        sid = lax.axis_index("s")
        gwid = sid * NC + cid
        ebase = gwid * EPT
        _fill(ones_v, CHUNKP, 16, 1.0)
        _fill(zb, ZR, 16, 0.0)
        _zero_acc(zb, acc, sid, zsem)
        plsc.subcore_barrier()

        for b in range(NBUF):
            pltpu.sync_copy(
                dst_hbm.at[pl.ds(ebase + b * CHUNKP, CHUNKP)], didxs[b])
            pltpu.async_copy(ones_v, acc.at[didxs[b]], ssems[b], add=True)

        @pl.loop(0, NCHUNK // NBUF - 1)
        def _(p):
            c0 = p * NBUF
            for b in range(NBUF):
                pltpu.make_async_copy(
                    ones_v, acc.at[didxs[b]], ssems[b]).wait()
                pltpu.sync_copy(
                    dst_hbm.at[pl.ds(ebase + (c0 + NBUF + b) * CHUNKP,
                                     CHUNKP)], didxs[b])
                pltpu.async_copy(ones_v, acc.at[didxs[b]], ssems[b], add=True)

        for b in range(NBUF):
            pltpu.make_async_copy(ones_v, acc.at[didxs[b]], ssems[b]).wait()

        plsc.subcore_barrier()
        _read_out(acc, out_hbm, cid, sid, zsem)

    return deg_kernel(dst)


def _edge_pipeline(g_hbm, sidx2, didx2, bufs, acc, gsems, ssems, nchunk):
    @pl.loop(0, nchunk // NBUF - 1)
    def _(p):
        c0 = p * NBUF
        for b in range(NBUF):
            pltpu.make_async_copy(
                g_hbm.at[sidx2.at[c0 + b]], bufs[b], gsems[b]).wait()
            pltpu.async_copy(
                bufs[b], acc.at[didx2.at[c0 + b]], ssems[b], add=True)
        for b in range(NBUF):
            pltpu.make_async_copy(
                bufs[b], acc.at[didx2.at[c0 + b]], ssems[b]).wait()
            pltpu.async_copy(
                g_hbm.at[sidx2.at[c0 + NBUF + b]], bufs[b], gsems[b])

    c0 = nchunk - NBUF
    for b in range(NBUF):
        pltpu.make_async_copy(
            g_hbm.at[sidx2.at[c0 + b]], bufs[b], gsems[b]).wait()
        pltpu.async_copy(
            bufs[b], acc.at[didx2.at[c0 + b]], ssems[b], add=True)
    for b in range(NBUF):
        pltpu.make_async_copy(
            bufs[b], acc.at[didx2.at[c0 + b]], ssems[b]).wait()


def _sc_edge_sum_feat(g0, g1, adj2):
    @functools.partial(
        pl.kernel,
        out_type=jax.ShapeDtypeStruct((NC, N, C), jnp.float32),
        mesh=_sc_mesh(),
        scratch_types=[
            pltpu.VMEM((NCHUNK2, CHUNKP), jnp.int32),
            pltpu.VMEM((NCHUNK2, CHUNKP), jnp.int32),
            [pltpu.VMEM((CHUNKP, C), jnp.float32)] * NBUF,
            pltpu.VMEM((ZR, C), jnp.float32),
            pltpu.VMEM_SHARED((N, C), jnp.float32),
            [pltpu.SemaphoreType.DMA] * NBUF,
            [pltpu.SemaphoreType.DMA] * NBUF,
            pltpu.SemaphoreType.DMA,
        ],
        compiler_params=pltpu.CompilerParams(use_tc_tiling_on_sc=False))
    def edge_kernel(g0_hbm, g1_hbm, adj_hbm, out_hbm,
                    sidx2, didx2, bufs, zb, acc, gsems, ssems, zsem):
        cid = lax.axis_index("c")
        sid = lax.axis_index("s")
        pltpu.sync_copy(adj_hbm.at[0, sid], sidx2)
        pltpu.sync_copy(adj_hbm.at[1, sid], didx2)
        _fill(zb, ZR, C, 0.0)
        _zero_acc(zb, acc, sid, zsem)
        plsc.subcore_barrier()

        @pl.when(cid == 0)
        def _():
            for b in range(NBUF):
                pltpu.async_copy(g0_hbm.at[sidx2.at[b]], bufs[b], gsems[b])
            _edge_pipeline(g0_hbm, sidx2, didx2, bufs, acc, gsems, ssems,
                           NCHUNK2)

        @pl.when(cid != 0)
        def _():
            for b in range(NBUF):
                pltpu.async_copy(g1_hbm.at[sidx2.at[b]], bufs[b], gsems[b])
            _edge_pipeline(g1_hbm, sidx2, didx2, bufs, acc, gsems, ssems,
                           NCHUNK2)

        plsc.subcore_barrier()
        _read_out(acc, out_hbm, cid, sid, zsem)

    return edge_kernel(g0, g1, adj2)


def _sc_edge_sum(g, adj3, width):
    @functools.partial(
        pl.kernel,
        out_type=jax.ShapeDtypeStruct((NC, N, width), jnp.float32),
        mesh=_sc_mesh(),
        scratch_types=[
            pltpu.VMEM((NCHUNK, CHUNKP), jnp.int32),
            pltpu.VMEM((NCHUNK, CHUNKP), jnp.int32),
            [pltpu.VMEM((CHUNKP, width), jnp.float32)] * NBUF,
            pltpu.VMEM((ZR, width), jnp.float32),
            pltpu.VMEM_SHARED((N, width), jnp.float32),
            [pltpu.SemaphoreType.DMA] * NBUF,
            [pltpu.SemaphoreType.DMA] * NBUF,
            pltpu.SemaphoreType.DMA,
        ],
        compiler_params=pltpu.CompilerParams(use_tc_tiling_on_sc=False))
    def edge_kernel(g_hbm, adj_hbm, out_hbm,
                    sidx2, didx2, bufs, zb, acc, gsems, ssems, zsem):
        cid = lax.axis_index("c")
        sid = lax.axis_index("s")
        gwid = sid * NC + cid
        pltpu.sync_copy(adj_hbm.at[0, gwid], sidx2)
        pltpu.sync_copy(adj_hbm.at[1, gwid], didx2)
        for b in range(NBUF):
            pltpu.async_copy(g_hbm.at[sidx2.at[b]], bufs[b], gsems[b])
        _fill(zb, ZR, width, 0.0)
        _zero_acc(zb, acc, sid, zsem)
        plsc.subcore_barrier()
        _edge_pipeline(g_hbm, sidx2, didx2, bufs, acc, gsems, ssems, NCHUNK)
        plsc.subcore_barrier()
        _read_out(acc, out_hbm, cid, sid, zsem)

    return edge_kernel(g, adj3)


def _deg_inv_sqrt(hist_blk):
    counts = hist_blk[0] + hist_blk[1]
    deg = jnp.sum(counts, axis=-1) * (1.0 / 16.0) + 1.0
    return lax.rsqrt(deg)[:, None]


def _tc_lin1(x, W1, hist):
    def lin1_kernel(x_ref, w_ref, hist_ref, g0_ref, g1_ref):
        h = jnp.dot(x_ref[...], w_ref[...],
                    preferred_element_type=jnp.float32)
        g = h * _deg_inv_sqrt(hist_ref[...])
        g0_ref[...] = g[:, :C]
        g1_ref[...] = g[:, C:]

    return pl.pallas_call(
        lin1_kernel,
        grid=(N // ROW_BLK,),
        in_specs=[
            pl.BlockSpec((ROW_BLK, F_IN), lambda i: (i, 0)),
            pl.BlockSpec((F_IN, H), lambda i: (0, 0)),
            pl.BlockSpec((NC, ROW_BLK, 16), lambda i: (0, i, 0)),
        ],
        out_specs=[
            pl.BlockSpec((ROW_BLK, C), lambda i: (i, 0)),
            pl.BlockSpec((ROW_BLK, C), lambda i: (i, 0)),
        ],
        out_shape=[
            jax.ShapeDtypeStruct((N, C), jnp.float32),
            jax.ShapeDtypeStruct((N, C), jnp.float32),
        ],
    )(x, W1, hist)


def _tc_layer2_in(S1s, g10, g11, hist, b1, W2):
    def l2_kernel(s_ref, g0_ref, g1_ref, hist_ref, b_ref, w_ref, o_ref):
        s = jnp.concatenate(
            [s_ref[0] + g0_ref[...], s_ref[1] + g1_ref[...]], axis=-1)
        dis = _deg_inv_sqrt(hist_ref[...])
        z = jnp.maximum(s * dis + b_ref[...], 0.0)
        o_ref[...] = jnp.dot(z, w_ref[...],
                             preferred_element_type=jnp.float32) * dis

    return pl.pallas_call(
        l2_kernel,
        grid=(N // ROW_BLK,),
        in_specs=[
            pl.BlockSpec((NC, ROW_BLK, C), lambda i: (0, i, 0)),
            pl.BlockSpec((ROW_BLK, C), lambda i: (i, 0)),
            pl.BlockSpec((ROW_BLK, C), lambda i: (i, 0)),
            pl.BlockSpec((NC, ROW_BLK, 16), lambda i: (0, i, 0)),
            pl.BlockSpec((H,), lambda i: (0,)),
            pl.BlockSpec((H, C), lambda i: (0, 0)),
        ],
        out_specs=pl.BlockSpec((ROW_BLK, C), lambda i: (i, 0)),
        out_shape=jax.ShapeDtypeStruct((N, C), jnp.float32),
    )(S1s, g10, g11, hist, b1, W2)


def _tc_out(S2, g2, hist, b2):
    def out_kernel(s_ref, g_ref, hist_ref, b_ref, o_ref):
        s = s_ref[0] + s_ref[1] + g_ref[...]
        dis = _deg_inv_sqrt(hist_ref[...])
        o_ref[...] = jax.nn.sigmoid(s * dis + b_ref[...])

    return pl.pallas_call(
        out_kernel,
        grid=(N // ROW_BLK,),
        in_specs=[
            pl.BlockSpec((NC, ROW_BLK, C), lambda i: (0, i, 0)),
            pl.BlockSpec((ROW_BLK, C), lambda i: (i, 0)),
            pl.BlockSpec((NC, ROW_BLK, 16), lambda i: (0, i, 0)),
            pl.BlockSpec((C,), lambda i: (0,)),
        ],
        out_specs=pl.BlockSpec((ROW_BLK, C), lambda i: (i, 0)),
        out_shape=jax.ShapeDtypeStruct((N, C), jnp.float32),
    )(S2, g2, hist, b2)


def kernel(x, adj, W1, b1, W2, b2):
    adj3 = adj.reshape(2, NW, NCHUNK, CHUNKP)
    adj2 = adj.reshape(2, NS, NCHUNK2, CHUNKP)
    hist = _sc_degree(adj[1])
    g10, g11 = _tc_lin1(x, W1, hist)
    S1s = _sc_edge_sum_feat(g10, g11, adj2)
    g2 = _tc_layer2_in(S1s, g10, g11, hist, b1, W2)
    S2 = _sc_edge_sum(g2, adj3, C)
    return _tc_out(S2, g2, hist, b2)


# speedup vs baseline: 1.1547x; 1.1547x over previous
import functools

import jax
import jax.numpy as jnp
from jax import lax
from jax.experimental import pallas as pl
from jax.experimental.pallas import tpu as pltpu
from jax.experimental.pallas import tpu_sc as plsc

N = 10000
E = 320000
F_IN = 128
H = 128
C = 64

NC = 2
NS = 16
NW = NC * NS
EPT = E // NW
CHUNKP = 80
NCHUNK = EPT // CHUNKP
EPT2 = E // NS
NCHUNK2 = EPT2 // CHUNKP
ROW0 = 624
TAIL0 = N - NS * ROW0
ZR = 40
NBUF = 5
ROW_BLK = 2000


def _sc_mesh():
    return plsc.VectorSubcoreMesh(
        core_axis_name="c", subcore_axis_name="s",
        num_cores=NC, num_subcores=NS)


def _fill(ref, rows, width, value):
    vec = jnp.full((16,), value, jnp.float32)

    @pl.loop(0, rows)
    def _(r):
        for c in range(0, width, 16):
            ref[r, pl.ds(c, 16)] = vec


def _zero_acc(zb, acc, sid, sem):
    base = sid * ROW0
    for z in range(16):
        pltpu.async_copy(zb, acc.at[pl.ds(base + z * ZR, ZR)], sem).wait()


def _read_out(acc, out_hbm, cid, sid, sem):
    base = sid * ROW0
    pltpu.async_copy(acc.at[pl.ds(base, ROW0)],
                     out_hbm.at[cid, pl.ds(base, ROW0)], sem).wait()

    @pl.when(sid == NS - 1)
    def _():
        t0 = NS * ROW0
        pltpu.async_copy(acc.at[pl.ds(t0, TAIL0)],
                         out_hbm.at[cid, pl.ds(t0, TAIL0)], sem).wait()


def _sc_degree(dst3):
    @functools.partial(
        pl.kernel,
        out_type=jax.ShapeDtypeStruct((NC, N, 16), jnp.float32),
        mesh=_sc_mesh(),
        scratch_types=[
            pltpu.VMEM((NCHUNK, CHUNKP), jnp.int32),
            pltpu.VMEM((CHUNKP, 16), jnp.float32),
            pltpu.VMEM((ZR, 16), jnp.float32),
            pltpu.VMEM_SHARED((N, 16), jnp.float32),
            [pltpu.SemaphoreType.DMA] * NBUF,
            pltpu.SemaphoreType.DMA,
        ])
    def deg_kernel(dst_hbm, out_hbm, didx2, ones_v, zb, acc, ssems, zsem):
        cid = lax.axis_index("c")
        sid = lax.axis_index("s")
        gwid = sid * NC + cid
        pltpu.sync_copy(dst_hbm.at[gwid], didx2)
        _fill(ones_v, CHUNKP, 16, 1.0)
        _fill(zb, ZR, 16, 0.0)
        _zero_acc(zb, acc, sid, zsem)
        plsc.subcore_barrier()

        for b in range(NBUF):
            pltpu.async_copy(ones_v, acc.at[didx2.at[b]], ssems[b], add=True)

        @pl.loop(0, NCHUNK // NBUF - 1)
        def _(p):
            c0 = p * NBUF
            for b in range(NBUF):
                pltpu.make_async_copy(
                    ones_v, acc.at[didx2.at[c0 + b]], ssems[b]).wait()
                pltpu.async_copy(
                    ones_v, acc.at[didx2.at[c0 + NBUF + b]], ssems[b],
                    add=True)

        for b in range(NBUF):
            pltpu.make_async_copy(
                ones_v, acc.at[didx2.at[NCHUNK - NBUF + b]], ssems[b]).wait()

        plsc.subcore_barrier()
        _read_out(acc, out_hbm, cid, sid, zsem)

    return deg_kernel(dst3)


def _edge_pipeline(g_hbm, sidx, didx, bufs, acc, gsems, ssems, nchunk):
    def s(c):
        return pl.ds(c * CHUNKP, CHUNKP)

    @pl.loop(0, nchunk // NBUF - 1)
    def _(p):
        c0 = p * NBUF
        for b in range(NBUF):
            pltpu.make_async_copy(
                g_hbm.at[sidx.at[s(c0 + b)]], bufs[b], gsems[b]).wait()
            pltpu.async_copy(
                bufs[b], acc.at[didx.at[s(c0 + b)]], ssems[b], add=True)
        for b in range(NBUF):
            pltpu.make_async_copy(
                bufs[b], acc.at[didx.at[s(c0 + b)]], ssems[b]).wait()
            pltpu.async_copy(
                g_hbm.at[sidx.at[s(c0 + NBUF + b)]], bufs[b], gsems[b])

    c0 = nchunk - NBUF
    for b in range(NBUF):
        pltpu.make_async_copy(
            g_hbm.at[sidx.at[s(c0 + b)]], bufs[b], gsems[b]).wait()
        pltpu.async_copy(
            bufs[b], acc.at[didx.at[s(c0 + b)]], ssems[b], add=True)
    for b in range(NBUF):
        pltpu.make_async_copy(
            bufs[b], acc.at[didx.at[s(c0 + b)]], ssems[b]).wait()


def _sc_edge_sum_feat(g0, g1, src, dst):
    @functools.partial(
        pl.kernel,
        out_type=jax.ShapeDtypeStruct((NC, N, C), jnp.float32),
        mesh=_sc_mesh(),
        scratch_types=[
            pltpu.VMEM((EPT2,), jnp.int32),
            pltpu.VMEM((EPT2,), jnp.int32),
            [pltpu.VMEM((CHUNKP, C), jnp.float32)] * NBUF,
            pltpu.VMEM((ZR, C), jnp.float32),
            pltpu.VMEM_SHARED((N, C), jnp.float32),
            [pltpu.SemaphoreType.DMA] * NBUF,
            [pltpu.SemaphoreType.DMA] * NBUF,
            pltpu.SemaphoreType.DMA,
        ],
        compiler_params=pltpu.CompilerParams(use_tc_tiling_on_sc=False))
    def edge_kernel(g0_hbm, g1_hbm, src_hbm, dst_hbm, out_hbm,
                    sidx, didx, bufs, zb, acc, gsems, ssems, zsem):
        cid = lax.axis_index("c")
        sid = lax.axis_index("s")
        ebase = sid * EPT2
        pltpu.sync_copy(src_hbm.at[pl.ds(ebase, EPT2)], sidx)
        pltpu.sync_copy(dst_hbm.at[pl.ds(ebase, EPT2)], didx)
        _fill(zb, ZR, C, 0.0)
        _zero_acc(zb, acc, sid, zsem)
        plsc.subcore_barrier()

        @pl.when(cid == 0)
        def _():
            for b in range(NBUF):
                pltpu.async_copy(
                    g0_hbm.at[sidx.at[pl.ds(b * CHUNKP, CHUNKP)]],
                    bufs[b], gsems[b])
            _edge_pipeline(g0_hbm, sidx, didx, bufs, acc, gsems, ssems,
                           NCHUNK2)

        @pl.when(cid != 0)
        def _():
            for b in range(NBUF):
                pltpu.async_copy(
                    g1_hbm.at[sidx.at[pl.ds(b * CHUNKP, CHUNKP)]],
                    bufs[b], gsems[b])
            _edge_pipeline(g1_hbm, sidx, didx, bufs, acc, gsems, ssems,
                           NCHUNK2)

        plsc.subcore_barrier()
        _read_out(acc, out_hbm, cid, sid, zsem)

    return edge_kernel(g0, g1, src, dst)


def _sc_edge_sum(g, src, dst, width):
    @functools.partial(
        pl.kernel,
        out_type=jax.ShapeDtypeStruct((NC, N, width), jnp.float32),
        mesh=_sc_mesh(),
        scratch_types=[
            pltpu.VMEM((EPT,), jnp.int32),
            pltpu.VMEM((EPT,), jnp.int32),
            [pltpu.VMEM((CHUNKP, width), jnp.float32)] * NBUF,
            pltpu.VMEM((ZR, width), jnp.float32),
            pltpu.VMEM_SHARED((N, width), jnp.float32),
            [pltpu.SemaphoreType.DMA] * NBUF,
            [pltpu.SemaphoreType.DMA] * NBUF,
            pltpu.SemaphoreType.DMA,
        ],
        compiler_params=pltpu.CompilerParams(use_tc_tiling_on_sc=False))
    def edge_kernel(g_hbm, src_hbm, dst_hbm, out_hbm,
                    sidx, didx, bufs, zb, acc, gsems, ssems, zsem):
        cid = lax.axis_index("c")
        sid = lax.axis_index("s")
        gwid = sid * NC + cid
        ebase = gwid * EPT
        pltpu.sync_copy(src_hbm.at[pl.ds(ebase, EPT)], sidx)
        pltpu.sync_copy(dst_hbm.at[pl.ds(ebase, EPT)], didx)
        for b in range(NBUF):
            pltpu.async_copy(g_hbm.at[sidx.at[pl.ds(b * CHUNKP, CHUNKP)]],
                             bufs[b], gsems[b])
        _fill(zb, ZR, width, 0.0)
        _zero_acc(zb, acc, sid, zsem)
        plsc.subcore_barrier()
        _edge_pipeline(g_hbm, sidx, didx, bufs, acc, gsems, ssems, NCHUNK)
        plsc.subcore_barrier()
        _read_out(acc, out_hbm, cid, sid, zsem)

    return edge_kernel(g, src, dst)


def _deg_inv_sqrt(hist_blk):
    counts = hist_blk[0] + hist_blk[1]
    deg = jnp.sum(counts, axis=-1) * (1.0 / 16.0) + 1.0
    return lax.rsqrt(deg)[:, None]


def _tc_lin1(x, W1, hist):
    def lin1_kernel(x_ref, w_ref, hist_ref, g0_ref, g1_ref):
        h = jnp.dot(x_ref[...], w_ref[...],
                    preferred_element_type=jnp.float32)
        g = h * _deg_inv_sqrt(hist_ref[...])
        g0_ref[...] = g[:, :C]
        g1_ref[...] = g[:, C:]

    return pl.pallas_call(
        lin1_kernel,
        grid=(N // ROW_BLK,),
        in_specs=[
            pl.BlockSpec((ROW_BLK, F_IN), lambda i: (i, 0)),
            pl.BlockSpec((F_IN, H), lambda i: (0, 0)),
            pl.BlockSpec((NC, ROW_BLK, 16), lambda i: (0, i, 0)),
        ],
        out_specs=[
            pl.BlockSpec((ROW_BLK, C), lambda i: (i, 0)),
            pl.BlockSpec((ROW_BLK, C), lambda i: (i, 0)),
        ],
        out_shape=[
            jax.ShapeDtypeStruct((N, C), jnp.float32),
            jax.ShapeDtypeStruct((N, C), jnp.float32),
        ],
    )(x, W1, hist)


def _tc_layer2_in(S1s, g10, g11, hist, b1, W2):
    def l2_kernel(s_ref, g0_ref, g1_ref, hist_ref, b_ref, w_ref, o_ref):
        s = jnp.concatenate(
            [s_ref[0] + g0_ref[...], s_ref[1] + g1_ref[...]], axis=-1)
        dis = _deg_inv_sqrt(hist_ref[...])
        z = jnp.maximum(s * dis + b_ref[...], 0.0)
        o_ref[...] = jnp.dot(z, w_ref[...],
                             preferred_element_type=jnp.float32) * dis

    return pl.pallas_call(
        l2_kernel,
        grid=(N // ROW_BLK,),
        in_specs=[
            pl.BlockSpec((NC, ROW_BLK, C), lambda i: (0, i, 0)),
            pl.BlockSpec((ROW_BLK, C), lambda i: (i, 0)),
            pl.BlockSpec((ROW_BLK, C), lambda i: (i, 0)),
            pl.BlockSpec((NC, ROW_BLK, 16), lambda i: (0, i, 0)),
            pl.BlockSpec((H,), lambda i: (0,)),
            pl.BlockSpec((H, C), lambda i: (0, 0)),
        ],
        out_specs=pl.BlockSpec((ROW_BLK, C), lambda i: (i, 0)),
        out_shape=jax.ShapeDtypeStruct((N, C), jnp.float32),
    )(S1s, g10, g11, hist, b1, W2)


def _tc_out(S2, g2, hist, b2):
    def out_kernel(s_ref, g_ref, hist_ref, b_ref, o_ref):
        s = s_ref[0] + s_ref[1] + g_ref[...]
        dis = _deg_inv_sqrt(hist_ref[...])
        o_ref[...] = jax.nn.sigmoid(s * dis + b_ref[...])

    return pl.pallas_call(
        out_kernel,
        grid=(N // ROW_BLK,),
        in_specs=[
            pl.BlockSpec((NC, ROW_BLK, C), lambda i: (0, i, 0)),
            pl.BlockSpec((ROW_BLK, C), lambda i: (i, 0)),
            pl.BlockSpec((NC, ROW_BLK, 16), lambda i: (0, i, 0)),
            pl.BlockSpec((C,), lambda i: (0,)),
        ],
        out_specs=pl.BlockSpec((ROW_BLK, C), lambda i: (i, 0)),
        out_shape=jax.ShapeDtypeStruct((N, C), jnp.float32),
    )(S2, g2, hist, b2)


def kernel(x, adj, W1, b1, W2, b2):
    src = adj[0]
    dst = adj[1]
    dst3 = dst.reshape(NW, NCHUNK, CHUNKP)
    hist = _sc_degree(dst3)
    g10, g11 = _tc_lin1(x, W1, hist)
    S1s = _sc_edge_sum_feat(g10, g11, src, dst)
    g2 = _tc_layer2_in(S1s, g10, g11, hist, b1, W2)
    S2 = _sc_edge_sum(g2, src, dst, C)
    return _tc_out(S2, g2, hist, b2)
